# baseline (speedup 1.0000x reference)
import functools

import jax
import jax.numpy as jnp
from jax import lax
from jax.experimental import pallas as pl
from jax.experimental.pallas import tpu as pltpu
from jax.experimental.pallas import tpu_sc as plsc

_COL_NUM = 12
_DMODEL = 16
_CAPACITY = 128
_BATCH = 8
_ROWS = 1024
_FEAT = _COL_NUM * _DMODEL
_HID = 32
_L = 16
_NCHUNK = _ROWS // _L


def _score_body(t_ref, w1t_ref, b1_ref, w2_ref, b2_ref, z_ref):
    t = t_ref[0]
    h = jnp.dot(t, w1t_ref[...], preferred_element_type=jnp.float32) + b1_ref[...]
    h = jnp.maximum(h, 0.0)
    s2 = jnp.dot(h, w2_ref[...], preferred_element_type=jnp.float32) + b2_ref[0, 0]
    mn = jnp.min(s2)
    mx = jnp.max(s2)
    z_ref[0] = (s2 - mn) / (mx - mn + 0.1) * 100.0


def _score(t, w1t, b1, w2, b2):
    return pl.pallas_call(
        _score_body,
        grid=(_BATCH,),
        in_specs=[
            pl.BlockSpec((1, _ROWS, _FEAT), lambda i: (i, 0, 0)),
            pl.BlockSpec((_FEAT, _HID), lambda i: (0, 0)),
            pl.BlockSpec((1, _HID), lambda i: (0, 0)),
            pl.BlockSpec((_HID, 1), lambda i: (0, 0)),
            pl.BlockSpec((1, 1), lambda i: (0, 0)),
        ],
        out_specs=pl.BlockSpec((1, _ROWS, 1), lambda i: (i, 0, 0)),
        out_shape=jax.ShapeDtypeStruct((_BATCH, _ROWS, 1), jnp.float32),
    )(t, w1t, b1, w2, b2)


_CCH = 64
_KC = 256
_NJ = _ROWS // _CCH


def _pos_body(zrow_ref, zcol_ref, p_ref):
    j = pl.program_id(1)
    zr_full = zrow_ref[0]
    zc = zcol_ref[0]
    cnt = jnp.zeros((_CCH, 1), jnp.int32)
    for k in range(_ROWS // _KC):
        zrk = zr_full[:, k * _KC:(k + 1) * _KC]
        gt = zrk > zc
        ic = lax.broadcasted_iota(jnp.int32, (_CCH, _KC), 0) + j * _CCH
        jc = lax.broadcasted_iota(jnp.int32, (_CCH, _KC), 1) + k * _KC
        tie = (zrk == zc) & (jc < ic)
        cnt = cnt + jnp.sum((gt | tie).astype(jnp.int32), axis=1, keepdims=True)
    p_ref[0] = cnt


def _positions(zrow, zcol):
    return pl.pallas_call(
        _pos_body,
        grid=(_BATCH, _NJ),
        in_specs=[
            pl.BlockSpec((1, 1, _ROWS), lambda i, j: (i, 0, 0)),
            pl.BlockSpec((1, _CCH, 1), lambda i, j: (i, j, 0)),
        ],
        out_specs=pl.BlockSpec((1, _CCH, 1), lambda i, j: (i, j, 0)),
        out_shape=jax.ShapeDtypeStruct((_BATCH, _ROWS, 1), jnp.int32),
    )(zrow, zcol)


def _pav_body(z_hbm, p_hbm, wc_hbm, wp_hbm, va_hbm, vcs_hbm, vws_hbm,
              z_v, p_v, s_v, blkf, blkcs, blkst, bid_v, wc_v, wp_v,
              oa_v, ocs_v, ows_v):
    cid = lax.axis_index("c")
    sid = lax.axis_index("s")
    lane = lax.iota(jnp.int32, _L)
    mask0 = lane == 0

    @pl.when(sid < _BATCH // 2)
    def _():
        row = cid * (_BATCH // 2) + sid
        pltpu.sync_copy(z_hbm.at[row], z_v)
        pltpu.sync_copy(p_hbm.at[row], p_v)
        pltpu.sync_copy(wc_hbm, wc_v)
        pltpu.sync_copy(wp_hbm, wp_v)

        def scatter_body(ci, _):
            sl = pl.ds(ci * _L, _L)
            plsc.store_scatter(s_v, [p_v[sl]], z_v[sl])
            blkst[sl] = jnp.zeros((_L,), jnp.int32)
            return 0

        lax.fori_loop(0, _NCHUNK, scatter_body, 0)
        blkst[pl.ds(_ROWS, _L)] = jnp.zeros((_L,), jnp.int32)

        def wsum_cf(a_i32, b_i32):
            a = a_i32.astype(jnp.float32)
            b = b_i32.astype(jnp.float32)
            return (b - a + 1.0) * (2.0 * _ROWS - a - b) * 0.5

        def pav_step(i, state):
            sp, tf, ts, tst = state
            ii = jnp.full((_L,), i, jnp.int32)
            cf0 = plsc.load_gather(s_v, [ii])
            cs0 = jnp.full((_L,), 1.0, jnp.float32)
            cst0 = ii

            def eval_viol(sp_, tf_, ts_, tst_, cf_, cs_, cst_):
                e = jnp.exp(cf_ - tf_)
                tw = wsum_cf(tst_, cst_ - 1)
                cw = wsum_cf(cst_, ii)
                viol = cs_ * e * tw > ts_ * cw
                return (sp_ > 0) & jnp.all(viol), e

            def cond(st):
                return st[0]

            def merge(st):
                _, e, sp_, tf_, ts_, tst_, cf_, cs_, cst_ = st
                mf, ms, mst = tf_, ts_ + e * cs_, tst_
                nidx = jnp.full((_L,), jnp.maximum(sp_ - 2, 0), jnp.int32)
                ntf = plsc.load_gather(blkf, [nidx])
                nts = plsc.load_gather(blkcs, [nidx])
                ntst = plsc.load_gather(blkst, [nidx])
                sp2 = sp_ - 1
                v2, e2 = eval_viol(sp2, ntf, nts, ntst, mf, ms, mst)
                return (v2, e2, sp2, ntf, nts, ntst, mf, ms, mst)

            v0, e0 = eval_viol(sp, tf, ts, tst, cf0, cs0, cst0)
            _, _, sp, tf, ts, tst, cf, cs, cst = lax.while_loop(
                cond, merge, (v0, e0, sp, tf, ts, tst, cf0, cs0, cst0))
            mpush = mask0 & jnp.full((_L,), sp > 0, jnp.bool_)
            spv = jnp.full((_L,), jnp.maximum(sp - 1, 0), jnp.int32)
            plsc.store_scatter(blkf, [spv], tf, mask=mpush)
            plsc.store_scatter(blkcs, [spv], ts, mask=mpush)
            plsc.store_scatter(blkst, [spv], tst, mask=mpush)
            return (sp + 1, cf, cs, cst)

        zero_f = jnp.zeros((_L,), jnp.float32)
        zero_i = jnp.zeros((_L,), jnp.int32)
        sp, tf, ts, tst = lax.fori_loop(
            0, _ROWS, pav_step, (0, zero_f, zero_f, zero_i))

        spv = jnp.full((_L,), sp - 1, jnp.int32)
        plsc.store_scatter(blkf, [spv], tf, mask=mask0)
        plsc.store_scatter(blkcs, [spv], ts, mask=mask0)
        plsc.store_scatter(blkst, [spv], tst, mask=mask0)
        spv = jnp.full((_L,), sp, jnp.int32)
        plsc.store_scatter(blkst, [spv], jnp.full((_L,), _ROWS, jnp.int32),
                           mask=mask0)

        def zero_body(ci, _):
            bid_v[pl.ds(ci * _L, _L)] = jnp.zeros((_L,), jnp.int32)
            return 0

        lax.fori_loop(0, _NCHUNK, zero_body, 0)

        ones = jnp.full((_L,), 1, jnp.int32)

        def mark_body(ci, _):
            idxs = ci * _L + lane
            st = blkst[pl.ds(ci * _L, _L)]
            plsc.store_scatter(bid_v, [st], ones, mask=idxs < sp)
            return 0

        lax.fori_loop(0, _NCHUNK, mark_body, 0)

        def cumsum_body(ci, carry):
            v = bid_v[pl.ds(ci * _L, _L)]
            cs_ = plsc.cumsum(v)
            bid_v[pl.ds(ci * _L, _L)] = cs_ + carry - 1
            return carry + jnp.full((_L,), jnp.max(cs_), jnp.int32)

        lax.fori_loop(0, _NCHUNK, cumsum_body, jnp.zeros((_L,), jnp.int32))

        def out_body(ci, _):
            sl = pl.ds(ci * _L, _L)
            pp = p_v[sl]
            b = plsc.load_gather(bid_v, [pp])
            st = plsc.load_gather(blkst, [b])
            en1 = plsc.load_gather(blkst, [b + 1])
            oa_v[sl] = plsc.load_gather(blkf, [b])
            ocs_v[sl] = plsc.load_gather(blkcs, [b])
            ows_v[sl] = (plsc.load_gather(wc_v, [en1 - 1])
                         - plsc.load_gather(wp_v, [st]))
            return 0

        lax.fori_loop(0, _NCHUNK, out_body, 0)

        pltpu.sync_copy(oa_v, va_hbm.at[row])
        pltpu.sync_copy(ocs_v, vcs_hbm.at[row])
        pltpu.sync_copy(ows_v, vws_hbm.at[row])


def _pav(z, p, wc, wprev):
    mesh = plsc.VectorSubcoreMesh(core_axis_name="c", subcore_axis_name="s")
    f = pl.kernel(
        _pav_body,
        compiler_params=pltpu.CompilerParams(needs_layout_passes=False),
        out_type=[
            jax.ShapeDtypeStruct((_BATCH, _ROWS), jnp.float32),
            jax.ShapeDtypeStruct((_BATCH, _ROWS), jnp.float32),
            jax.ShapeDtypeStruct((_BATCH, _ROWS), jnp.float32),
        ],
        mesh=mesh,
        scratch_types=[
            pltpu.VMEM((_ROWS,), jnp.float32),
            pltpu.VMEM((_ROWS,), jnp.int32),
            pltpu.VMEM((_ROWS,), jnp.float32),
            pltpu.VMEM((_ROWS,), jnp.float32),
            pltpu.VMEM((_ROWS,), jnp.float32),
            pltpu.VMEM((_ROWS + _L,), jnp.int32),
            pltpu.VMEM((_ROWS,), jnp.int32),
            pltpu.VMEM((_ROWS,), jnp.float32),
            pltpu.VMEM((_ROWS,), jnp.float32),
            pltpu.VMEM((_ROWS,), jnp.float32),
            pltpu.VMEM((_ROWS,), jnp.float32),
            pltpu.VMEM((_ROWS,), jnp.float32),
        ],
    )
    return f(z, p, wc, wprev)


def _final_body(z_ref, a_ref, cs_ref, ws_ref, o_ref):
    z = z_ref[...]
    lse_s = a_ref[...] + jnp.log(jnp.maximum(cs_ref[...], 1e-38))
    v = lse_s - jnp.log(jnp.maximum(ws_ref[...], 1e-9))
    r = jnp.exp(z - v)
    other = r % 128.0
    o_ref[...] = (r - other) / _CAPACITY + 1.0


def _final(z, va, vcs, vws):
    return pl.pallas_call(
        _final_body,
        out_shape=jax.ShapeDtypeStruct((_BATCH, _ROWS), jnp.float32),
    )(z, va, vcs, vws)


def kernel(table, W1, b1, W2, b2):
    t = table.reshape(_BATCH, _ROWS, _FEAT)
    w_lin = jnp.arange(_ROWS, 0, -1).astype(jnp.float32)
    wc = jnp.cumsum(w_lin)
    wprev = jnp.concatenate([jnp.zeros((1,), jnp.float32), wc[:-1]])
    zcol = _score(t, W1.T, b1.reshape(1, _HID), W2.T, b2.reshape(1, 1))
    z = zcol.reshape(_BATCH, _ROWS)
    zrow = z.reshape(_BATCH, 1, _ROWS)
    p = _positions(zrow, zcol).reshape(_BATCH, _ROWS)
    va, vcs, vws = _pav(z, p, wc, wprev)
    out = _final(z, va, vcs, vws)
    return out.reshape(_BATCH, _ROWS, 1)


# speedup vs baseline: 1.2194x; 1.2194x over previous
import functools

import jax
import jax.numpy as jnp
from jax import lax
from jax.experimental import pallas as pl
from jax.experimental.pallas import tpu as pltpu
from jax.experimental.pallas import tpu_sc as plsc

_COL_NUM = 12
_DMODEL = 16
_CAPACITY = 128
_BATCH = 8
_ROWS = 1024
_FEAT = _COL_NUM * _DMODEL
_HID = 32
_L = 16
_NCHUNK = _ROWS // _L


def _score_body(t_ref, w1t_ref, b1_ref, w2_ref, b2_ref, z_ref):
    t = t_ref[0]
    h = jnp.dot(t, w1t_ref[...], preferred_element_type=jnp.float32) + b1_ref[...]
    h = jnp.maximum(h, 0.0)
    s2 = jnp.dot(h, w2_ref[...], preferred_element_type=jnp.float32) + b2_ref[0, 0]
    mn = jnp.min(s2)
    mx = jnp.max(s2)
    z_ref[0] = (s2 - mn) / (mx - mn + 0.1) * 100.0


def _score(t, w1t, b1, w2, b2):
    return pl.pallas_call(
        _score_body,
        grid=(_BATCH,),
        in_specs=[
            pl.BlockSpec((1, _ROWS, _FEAT), lambda i: (i, 0, 0)),
            pl.BlockSpec((_FEAT, _HID), lambda i: (0, 0)),
            pl.BlockSpec((1, _HID), lambda i: (0, 0)),
            pl.BlockSpec((_HID, 1), lambda i: (0, 0)),
            pl.BlockSpec((1, 1), lambda i: (0, 0)),
        ],
        out_specs=pl.BlockSpec((1, _ROWS, 1), lambda i: (i, 0, 0)),
        out_shape=jax.ShapeDtypeStruct((_BATCH, _ROWS, 1), jnp.float32),
    )(t, w1t, b1, w2, b2)


_CCH = 128
_NJ = _ROWS // _CCH


def _pos_body(zrow_ref, zcol_ref, p_ref):
    j = pl.program_id(1)
    zr = zrow_ref[0]
    zc = zcol_ref[0]
    gt = zr > zc
    ic = lax.broadcasted_iota(jnp.int32, (_CCH, _ROWS), 0) + j * _CCH
    jc = lax.broadcasted_iota(jnp.int32, (_CCH, _ROWS), 1)
    tie = (zr == zc) & (jc < ic)
    m = jnp.where(gt | tie, 1.0, 0.0)
    cnt = jnp.dot(m, jnp.ones((_ROWS, 1), jnp.float32),
                  preferred_element_type=jnp.float32)
    p_ref[0] = cnt.astype(jnp.int32)


def _positions(zrow, zcol):
    return pl.pallas_call(
        _pos_body,
        grid=(_BATCH, _NJ),
        in_specs=[
            pl.BlockSpec((1, 1, _ROWS), lambda i, j: (i, 0, 0)),
            pl.BlockSpec((1, _CCH, 1), lambda i, j: (i, j, 0)),
        ],
        out_specs=pl.BlockSpec((1, _CCH, 1), lambda i, j: (i, j, 0)),
        out_shape=jax.ShapeDtypeStruct((_BATCH, _ROWS, 1), jnp.int32),
    )(zrow, zcol)


def _pav_body(z_hbm, p_hbm, wc_hbm, wp_hbm, va_hbm, vcs_hbm, vws_hbm,
              z_v, p_v, s_v, blkf, blkcs, blkst, bid_v, wc_v, wp_v,
              oa_v, ocs_v, ows_v):
    cid = lax.axis_index("c")
    sid = lax.axis_index("s")
    lane = lax.iota(jnp.int32, _L)
    mask0 = lane == 0

    @pl.when(sid < _BATCH // 2)
    def _():
        row = cid * (_BATCH // 2) + sid
        pltpu.sync_copy(z_hbm.at[row], z_v)
        pltpu.sync_copy(p_hbm.at[row], p_v)
        pltpu.sync_copy(wc_hbm, wc_v)
        pltpu.sync_copy(wp_hbm, wp_v)

        def scatter_body(ci, _):
            sl = pl.ds(ci * _L, _L)
            plsc.store_scatter(s_v, [p_v[sl]], z_v[sl])
            blkst[sl] = jnp.zeros((_L,), jnp.int32)
            return 0

        lax.fori_loop(0, _NCHUNK, scatter_body, 0)
        blkst[pl.ds(_ROWS, _L)] = jnp.zeros((_L,), jnp.int32)

        def wsum_cf(a_i32, b_i32):
            a = a_i32.astype(jnp.float32)
            b = b_i32.astype(jnp.float32)
            return (b - a + 1.0) * (2.0 * _ROWS - a - b) * 0.5

        def pav_step(i, state):
            sp, tf, ts, tst = state
            ii = jnp.full((_L,), i, jnp.int32)
            cf0 = plsc.load_gather(s_v, [ii])
            cs0 = jnp.full((_L,), 1.0, jnp.float32)
            cst0 = ii

            def eval_viol(sp_, tf_, ts_, tst_, cf_, cs_, cst_):
                e = jnp.exp(cf_ - tf_)
                tw = wsum_cf(tst_, cst_ - 1)
                cw = wsum_cf(cst_, ii)
                viol = cs_ * e * tw > ts_ * cw
                return (sp_ > 0) & jnp.all(viol), e

            def cond(st):
                return st[0]

            def merge(st):
                _, e, sp_, tf_, ts_, tst_, cf_, cs_, cst_ = st
                mf, ms, mst = tf_, ts_ + e * cs_, tst_
                nidx = jnp.full((_L,), jnp.maximum(sp_ - 2, 0), jnp.int32)
                ntf = plsc.load_gather(blkf, [nidx])
                nts = plsc.load_gather(blkcs, [nidx])
                ntst = plsc.load_gather(blkst, [nidx])
                sp2 = sp_ - 1
                v2, e2 = eval_viol(sp2, ntf, nts, ntst, mf, ms, mst)
                return (v2, e2, sp2, ntf, nts, ntst, mf, ms, mst)

            v0, e0 = eval_viol(sp, tf, ts, tst, cf0, cs0, cst0)
            _, _, sp, tf, ts, tst, cf, cs, cst = lax.while_loop(
                cond, merge, (v0, e0, sp, tf, ts, tst, cf0, cs0, cst0))
            mpush = mask0 & jnp.full((_L,), sp > 0, jnp.bool_)
            spv = jnp.full((_L,), jnp.maximum(sp - 1, 0), jnp.int32)
            plsc.store_scatter(blkf, [spv], tf, mask=mpush)
            plsc.store_scatter(blkcs, [spv], ts, mask=mpush)
            plsc.store_scatter(blkst, [spv], tst, mask=mpush)
            return (sp + 1, cf, cs, cst)

        zero_f = jnp.zeros((_L,), jnp.float32)
        zero_i = jnp.zeros((_L,), jnp.int32)
        sp, tf, ts, tst = lax.fori_loop(
            0, _ROWS, pav_step, (0, zero_f, zero_f, zero_i))

        spv = jnp.full((_L,), sp - 1, jnp.int32)
        plsc.store_scatter(blkf, [spv], tf, mask=mask0)
        plsc.store_scatter(blkcs, [spv], ts, mask=mask0)
        plsc.store_scatter(blkst, [spv], tst, mask=mask0)
        spv = jnp.full((_L,), sp, jnp.int32)
        plsc.store_scatter(blkst, [spv], jnp.full((_L,), _ROWS, jnp.int32),
                           mask=mask0)

        def zero_body(ci, _):
            bid_v[pl.ds(ci * _L, _L)] = jnp.zeros((_L,), jnp.int32)
            return 0

        lax.fori_loop(0, _NCHUNK, zero_body, 0)

        ones = jnp.full((_L,), 1, jnp.int32)

        def mark_body(ci, _):
            idxs = ci * _L + lane
            st = blkst[pl.ds(ci * _L, _L)]
            plsc.store_scatter(bid_v, [st], ones, mask=idxs < sp)
            return 0

        lax.fori_loop(0, _NCHUNK, mark_body, 0)

        def cumsum_body(ci, carry):
            v = bid_v[pl.ds(ci * _L, _L)]
            cs_ = plsc.cumsum(v)
            bid_v[pl.ds(ci * _L, _L)] = cs_ + carry - 1
            return carry + jnp.full((_L,), jnp.max(cs_), jnp.int32)

        lax.fori_loop(0, _NCHUNK, cumsum_body, jnp.zeros((_L,), jnp.int32))

        def out_body(ci, _):
            sl = pl.ds(ci * _L, _L)
            pp = p_v[sl]
            b = plsc.load_gather(bid_v, [pp])
            st = plsc.load_gather(blkst, [b])
            en1 = plsc.load_gather(blkst, [b + 1])
            oa_v[sl] = plsc.load_gather(blkf, [b])
            ocs_v[sl] = plsc.load_gather(blkcs, [b])
            ows_v[sl] = (plsc.load_gather(wc_v, [en1 - 1])
                         - plsc.load_gather(wp_v, [st]))
            return 0

        lax.fori_loop(0, _NCHUNK, out_body, 0)

        pltpu.sync_copy(oa_v, va_hbm.at[row])
        pltpu.sync_copy(ocs_v, vcs_hbm.at[row])
        pltpu.sync_copy(ows_v, vws_hbm.at[row])


def _pav(z, p, wc, wprev):
    mesh = plsc.VectorSubcoreMesh(core_axis_name="c", subcore_axis_name="s")
    f = pl.kernel(
        _pav_body,
        compiler_params=pltpu.CompilerParams(needs_layout_passes=False),
        out_type=[
            jax.ShapeDtypeStruct((_BATCH, _ROWS), jnp.float32),
            jax.ShapeDtypeStruct((_BATCH, _ROWS), jnp.float32),
            jax.ShapeDtypeStruct((_BATCH, _ROWS), jnp.float32),
        ],
        mesh=mesh,
        scratch_types=[
            pltpu.VMEM((_ROWS,), jnp.float32),
            pltpu.VMEM((_ROWS,), jnp.int32),
            pltpu.VMEM((_ROWS,), jnp.float32),
            pltpu.VMEM((_ROWS,), jnp.float32),
            pltpu.VMEM((_ROWS,), jnp.float32),
            pltpu.VMEM((_ROWS + _L,), jnp.int32),
            pltpu.VMEM((_ROWS,), jnp.int32),
            pltpu.VMEM((_ROWS,), jnp.float32),
            pltpu.VMEM((_ROWS,), jnp.float32),
            pltpu.VMEM((_ROWS,), jnp.float32),
            pltpu.VMEM((_ROWS,), jnp.float32),
            pltpu.VMEM((_ROWS,), jnp.float32),
        ],
    )
    return f(z, p, wc, wprev)


def _final_body(z_ref, a_ref, cs_ref, ws_ref, o_ref):
    z = z_ref[...]
    lse_s = a_ref[...] + jnp.log(jnp.maximum(cs_ref[...], 1e-38))
    v = lse_s - jnp.log(jnp.maximum(ws_ref[...], 1e-9))
    r = jnp.exp(z - v)
    other = r % 128.0
    o_ref[...] = (r - other) / _CAPACITY + 1.0


def _final(z, va, vcs, vws):
    return pl.pallas_call(
        _final_body,
        out_shape=jax.ShapeDtypeStruct((_BATCH, _ROWS), jnp.float32),
    )(z, va, vcs, vws)


def kernel(table, W1, b1, W2, b2):
    t = table.reshape(_BATCH, _ROWS, _FEAT)
    w_lin = jnp.arange(_ROWS, 0, -1).astype(jnp.float32)
    wc = jnp.cumsum(w_lin)
    wprev = jnp.concatenate([jnp.zeros((1,), jnp.float32), wc[:-1]])
    zcol = _score(t, W1.T, b1.reshape(1, _HID), W2.T, b2.reshape(1, 1))
    z = zcol.reshape(_BATCH, _ROWS)
    zrow = z.reshape(_BATCH, 1, _ROWS)
    p = _positions(zrow, zcol).reshape(_BATCH, _ROWS)
    va, vcs, vws = _pav(z, p, wc, wprev)
    out = _final(z, va, vcs, vws)
    return out.reshape(_BATCH, _ROWS, 1)


# speedup vs baseline: 1.3841x; 1.1351x over previous
import functools

import jax
import jax.numpy as jnp
from jax import lax
from jax.experimental import pallas as pl
from jax.experimental.pallas import tpu as pltpu
from jax.experimental.pallas import tpu_sc as plsc

_COL_NUM = 12
_DMODEL = 16
_CAPACITY = 128
_BATCH = 8
_ROWS = 1024
_FEAT = _COL_NUM * _DMODEL
_HID = 32
_L = 16
_NCHUNK = _ROWS // _L


def _score_body(t_ref, w1t_ref, b1_ref, w2_ref, b2_ref, z_ref):
    t = t_ref[0]
    h = jnp.dot(t, w1t_ref[...], preferred_element_type=jnp.float32) + b1_ref[...]
    h = jnp.maximum(h, 0.0)
    s2 = jnp.dot(h, w2_ref[...], preferred_element_type=jnp.float32) + b2_ref[0, 0]
    mn = jnp.min(s2)
    mx = jnp.max(s2)
    z_ref[0] = (s2 - mn) / (mx - mn + 0.1) * 100.0


def _score(t, w1t, b1, w2, b2):
    return pl.pallas_call(
        _score_body,
        grid=(_BATCH,),
        in_specs=[
            pl.BlockSpec((1, _ROWS, _FEAT), lambda i: (i, 0, 0)),
            pl.BlockSpec((_FEAT, _HID), lambda i: (0, 0)),
            pl.BlockSpec((1, _HID), lambda i: (0, 0)),
            pl.BlockSpec((_HID, 1), lambda i: (0, 0)),
            pl.BlockSpec((1, 1), lambda i: (0, 0)),
        ],
        out_specs=pl.BlockSpec((1, _ROWS, 1), lambda i: (i, 0, 0)),
        out_shape=jax.ShapeDtypeStruct((_BATCH, _ROWS, 1), jnp.float32),
    )(t, w1t, b1, w2, b2)


_CCH = 256
_NJ = _ROWS // _CCH


def _pos_body(zrow_ref, zcol_ref, p_ref):
    j = pl.program_id(1)
    zr = zrow_ref[0]
    zc = zcol_ref[0]
    gt = zr > zc
    ic = lax.broadcasted_iota(jnp.int32, (_CCH, _ROWS), 0) + j * _CCH
    jc = lax.broadcasted_iota(jnp.int32, (_CCH, _ROWS), 1)
    tie = (zr == zc) & (jc < ic)
    m = jnp.where(gt | tie, 1.0, 0.0)
    cnt = jnp.dot(m, jnp.ones((_ROWS, 1), jnp.float32),
                  preferred_element_type=jnp.float32)
    p_ref[0] = cnt.astype(jnp.int32)


def _positions(zrow, zcol):
    return pl.pallas_call(
        _pos_body,
        grid=(_BATCH, _NJ),
        in_specs=[
            pl.BlockSpec((1, 1, _ROWS), lambda i, j: (i, 0, 0)),
            pl.BlockSpec((1, _CCH, 1), lambda i, j: (i, j, 0)),
        ],
        out_specs=pl.BlockSpec((1, _CCH, 1), lambda i, j: (i, j, 0)),
        out_shape=jax.ShapeDtypeStruct((_BATCH, _ROWS, 1), jnp.int32),
    )(zrow, zcol)


def _pav_body(z_hbm, p_hbm, wc_hbm, wp_hbm, va_hbm, vcs_hbm, vws_hbm,
              z_v, p_v, s_v, blkf, blkcs, blkst, bid_v, wc_v, wp_v,
              oa_v, ocs_v, ows_v):
    cid = lax.axis_index("c")
    sid = lax.axis_index("s")
    lane = lax.iota(jnp.int32, _L)
    mask0 = lane == 0

    @pl.when(sid < _BATCH // 2)
    def _():
        row = cid * (_BATCH // 2) + sid
        pltpu.sync_copy(z_hbm.at[row], z_v)
        pltpu.sync_copy(p_hbm.at[row], p_v)
        pltpu.sync_copy(wc_hbm, wc_v)
        pltpu.sync_copy(wp_hbm, wp_v)

        def scatter_body(ci, _):
            sl = pl.ds(ci * _L, _L)
            plsc.store_scatter(s_v, [p_v[sl]], z_v[sl])
            blkst[sl] = jnp.zeros((_L,), jnp.int32)
            return 0

        lax.fori_loop(0, _NCHUNK, scatter_body, 0)
        blkst[pl.ds(_ROWS, _L)] = jnp.zeros((_L,), jnp.int32)

        def wsum_cf(a_i32, b_i32):
            a = a_i32.astype(jnp.float32)
            b = b_i32.astype(jnp.float32)
            return (b - a + 1.0) * (2.0 * _ROWS - a - b) * 0.5

        def pav_step(i, state):
            sp, tf, ts, tst, cf0 = state
            ii = jnp.full((_L,), i, jnp.int32)
            cfn = plsc.load_gather(s_v, [ii + 1])
            cs0 = jnp.full((_L,), 1.0, jnp.float32)
            cst0 = ii

            def eval_viol(sp_, tf_, ts_, tst_, cf_, cs_, cst_):
                e = jnp.exp(cf_ - tf_)
                tw = wsum_cf(tst_, cst_ - 1)
                cw = wsum_cf(cst_, ii)
                d = cs_ * e * tw - ts_ * cw
                return (sp_ > 0) & (d[0] > 0.0), e

            def cond(st):
                return st[0]

            def merge(st):
                _, e, sp_, tf_, ts_, tst_, cf_, cs_, cst_ = st
                mf, ms, mst = tf_, ts_ + e * cs_, tst_
                nidx = jnp.full((_L,), jnp.maximum(sp_ - 2, 0), jnp.int32)
                ntf = plsc.load_gather(blkf, [nidx])
                nts = plsc.load_gather(blkcs, [nidx])
                ntst = plsc.load_gather(blkst, [nidx])
                sp2 = sp_ - 1
                v2, e2 = eval_viol(sp2, ntf, nts, ntst, mf, ms, mst)
                return (v2, e2, sp2, ntf, nts, ntst, mf, ms, mst)

            v0, e0 = eval_viol(sp, tf, ts, tst, cf0, cs0, cst0)
            _, _, sp, tf, ts, tst, cf, cs, cst = lax.while_loop(
                cond, merge, (v0, e0, sp, tf, ts, tst, cf0, cs0, cst0))
            mpush = mask0 & jnp.full((_L,), sp > 0, jnp.bool_)
            spv = jnp.full((_L,), jnp.maximum(sp - 1, 0), jnp.int32)
            plsc.store_scatter(blkf, [spv], tf, mask=mpush)
            plsc.store_scatter(blkcs, [spv], ts, mask=mpush)
            plsc.store_scatter(blkst, [spv], tst, mask=mpush)
            return (sp + 1, cf, cs, cst, cfn)

        zero_f = jnp.zeros((_L,), jnp.float32)
        zero_i = jnp.zeros((_L,), jnp.int32)
        cf_first = plsc.load_gather(s_v, [jnp.zeros((_L,), jnp.int32)])
        sp, tf, ts, tst, _ = lax.fori_loop(
            0, _ROWS, pav_step, (0, zero_f, zero_f, zero_i, cf_first))

        spv = jnp.full((_L,), sp - 1, jnp.int32)
        plsc.store_scatter(blkf, [spv], tf, mask=mask0)
        plsc.store_scatter(blkcs, [spv], ts, mask=mask0)
        plsc.store_scatter(blkst, [spv], tst, mask=mask0)
        spv = jnp.full((_L,), sp, jnp.int32)
        plsc.store_scatter(blkst, [spv], jnp.full((_L,), _ROWS, jnp.int32),
                           mask=mask0)

        def zero_body(ci, _):
            bid_v[pl.ds(ci * _L, _L)] = jnp.zeros((_L,), jnp.int32)
            return 0

        lax.fori_loop(0, _NCHUNK, zero_body, 0)

        ones = jnp.full((_L,), 1, jnp.int32)

        def mark_body(ci, _):
            idxs = ci * _L + lane
            st = blkst[pl.ds(ci * _L, _L)]
            plsc.store_scatter(bid_v, [st], ones, mask=idxs < sp)
            return 0

        lax.fori_loop(0, _NCHUNK, mark_body, 0)

        def cumsum_body(ci, carry):
            v = bid_v[pl.ds(ci * _L, _L)]
            cs_ = plsc.cumsum(v)
            bid_v[pl.ds(ci * _L, _L)] = cs_ + carry - 1
            return carry + jnp.full((_L,), jnp.max(cs_), jnp.int32)

        lax.fori_loop(0, _NCHUNK, cumsum_body, jnp.zeros((_L,), jnp.int32))

        def out_body(ci, _):
            sl = pl.ds(ci * _L, _L)
            pp = p_v[sl]
            b = plsc.load_gather(bid_v, [pp])
            st = plsc.load_gather(blkst, [b])
            en1 = plsc.load_gather(blkst, [b + 1])
            oa_v[sl] = plsc.load_gather(blkf, [b])
            ocs_v[sl] = plsc.load_gather(blkcs, [b])
            ows_v[sl] = (plsc.load_gather(wc_v, [en1 - 1])
                         - plsc.load_gather(wp_v, [st]))
            return 0

        lax.fori_loop(0, _NCHUNK, out_body, 0)

        pltpu.sync_copy(oa_v, va_hbm.at[row])
        pltpu.sync_copy(ocs_v, vcs_hbm.at[row])
        pltpu.sync_copy(ows_v, vws_hbm.at[row])


def _pav(z, p, wc, wprev):
    mesh = plsc.VectorSubcoreMesh(core_axis_name="c", subcore_axis_name="s")
    f = pl.kernel(
        _pav_body,
        compiler_params=pltpu.CompilerParams(needs_layout_passes=False),
        out_type=[
            jax.ShapeDtypeStruct((_BATCH, _ROWS), jnp.float32),
            jax.ShapeDtypeStruct((_BATCH, _ROWS), jnp.float32),
            jax.ShapeDtypeStruct((_BATCH, _ROWS), jnp.float32),
        ],
        mesh=mesh,
        scratch_types=[
            pltpu.VMEM((_ROWS,), jnp.float32),
            pltpu.VMEM((_ROWS,), jnp.int32),
            pltpu.VMEM((_ROWS + _L,), jnp.float32),
            pltpu.VMEM((_ROWS,), jnp.float32),
            pltpu.VMEM((_ROWS,), jnp.float32),
            pltpu.VMEM((_ROWS + _L,), jnp.int32),
            pltpu.VMEM((_ROWS,), jnp.int32),
            pltpu.VMEM((_ROWS,), jnp.float32),
            pltpu.VMEM((_ROWS,), jnp.float32),
            pltpu.VMEM((_ROWS,), jnp.float32),
            pltpu.VMEM((_ROWS,), jnp.float32),
            pltpu.VMEM((_ROWS,), jnp.float32),
        ],
    )
    return f(z, p, wc, wprev)


def _final_body(z_ref, a_ref, cs_ref, ws_ref, o_ref):
    z = z_ref[...]
    lse_s = a_ref[...] + jnp.log(jnp.maximum(cs_ref[...], 1e-38))
    v = lse_s - jnp.log(jnp.maximum(ws_ref[...], 1e-9))
    r = jnp.exp(z - v)
    other = r % 128.0
    o_ref[...] = (r - other) / _CAPACITY + 1.0


def _final(z, va, vcs, vws):
    return pl.pallas_call(
        _final_body,
        out_shape=jax.ShapeDtypeStruct((_BATCH, _ROWS), jnp.float32),
    )(z, va, vcs, vws)


def kernel(table, W1, b1, W2, b2):
    t = table.reshape(_BATCH, _ROWS, _FEAT)
    w_lin = jnp.arange(_ROWS, 0, -1).astype(jnp.float32)
    wc = jnp.cumsum(w_lin)
    wprev = jnp.concatenate([jnp.zeros((1,), jnp.float32), wc[:-1]])
    zcol = _score(t, W1.T, b1.reshape(1, _HID), W2.T, b2.reshape(1, 1))
    z = zcol.reshape(_BATCH, _ROWS)
    zrow = z.reshape(_BATCH, 1, _ROWS)
    p = _positions(zrow, zcol).reshape(_BATCH, _ROWS)
    va, vcs, vws = _pav(z, p, wc, wprev)
    out = _final(z, va, vcs, vws)
    return out.reshape(_BATCH, _ROWS, 1)


# speedup vs baseline: 1.4979x; 1.0822x over previous
import functools

import jax
import jax.numpy as jnp
from jax import lax
from jax.experimental import pallas as pl
from jax.experimental.pallas import tpu as pltpu
from jax.experimental.pallas import tpu_sc as plsc

_COL_NUM = 12
_DMODEL = 16
_CAPACITY = 128
_BATCH = 8
_ROWS = 1024
_FEAT = _COL_NUM * _DMODEL
_HID = 32
_L = 16
_NCHUNK = _ROWS // _L


def _score_body(t_ref, w1t_ref, b1_ref, w2_ref, b2_ref, z_ref):
    t = t_ref[0]
    h = jnp.dot(t, w1t_ref[...], preferred_element_type=jnp.float32) + b1_ref[...]
    h = jnp.maximum(h, 0.0)
    s2 = jnp.dot(h, w2_ref[...], preferred_element_type=jnp.float32) + b2_ref[0, 0]
    mn = jnp.min(s2)
    mx = jnp.max(s2)
    z_ref[0] = (s2 - mn) / (mx - mn + 0.1) * 100.0


def _score(t, w1t, b1, w2, b2):
    return pl.pallas_call(
        _score_body,
        grid=(_BATCH,),
        in_specs=[
            pl.BlockSpec((1, _ROWS, _FEAT), lambda i: (i, 0, 0)),
            pl.BlockSpec((_FEAT, _HID), lambda i: (0, 0)),
            pl.BlockSpec((1, _HID), lambda i: (0, 0)),
            pl.BlockSpec((_HID, 1), lambda i: (0, 0)),
            pl.BlockSpec((1, 1), lambda i: (0, 0)),
        ],
        out_specs=pl.BlockSpec((1, _ROWS, 1), lambda i: (i, 0, 0)),
        out_shape=jax.ShapeDtypeStruct((_BATCH, _ROWS, 1), jnp.float32),
    )(t, w1t, b1, w2, b2)


_CCH = 512
_NJ = _ROWS // _CCH


def _pos_body(zrow_ref, zcol_ref, p_ref):
    j = pl.program_id(1)
    zr = zrow_ref[0]
    zc = zcol_ref[0]
    gt = zr > zc
    ic = lax.broadcasted_iota(jnp.int32, (_CCH, _ROWS), 0) + j * _CCH
    jc = lax.broadcasted_iota(jnp.int32, (_CCH, _ROWS), 1)
    tie = (zr == zc) & (jc < ic)
    m = jnp.where(gt | tie, 1.0, 0.0)
    cnt = jnp.dot(m, jnp.ones((_ROWS, 1), jnp.float32),
                  preferred_element_type=jnp.float32)
    p_ref[0] = cnt.astype(jnp.int32)


def _positions(zrow, zcol):
    return pl.pallas_call(
        _pos_body,
        grid=(_BATCH, _NJ),
        in_specs=[
            pl.BlockSpec((1, 1, _ROWS), lambda i, j: (i, 0, 0)),
            pl.BlockSpec((1, _CCH, 1), lambda i, j: (i, j, 0)),
        ],
        out_specs=pl.BlockSpec((1, _CCH, 1), lambda i, j: (i, j, 0)),
        out_shape=jax.ShapeDtypeStruct((_BATCH, _ROWS, 1), jnp.int32),
    )(zrow, zcol)


def _pav_body(z_hbm, p_hbm, wc_hbm, wp_hbm, va_hbm, vcs_hbm, vws_hbm,
              z_v, p_v, s_v, blkf, blkcs, blkst, bid_v, wc_v, wp_v,
              oa_v, ocs_v, ows_v, sem):
    cid = lax.axis_index("c")
    sid = lax.axis_index("s")
    lane = lax.iota(jnp.int32, _L)
    mask0 = lane == 0

    @pl.when(sid < _BATCH // 2)
    def _():
        row = cid * (_BATCH // 2) + sid
        c1 = pltpu.async_copy(z_hbm.at[row], z_v, sem)
        c2 = pltpu.async_copy(p_hbm.at[row], p_v, sem)
        c3 = pltpu.async_copy(wc_hbm, wc_v, sem)
        c4 = pltpu.async_copy(wp_hbm, wp_v, sem)
        c1.wait(); c2.wait(); c3.wait(); c4.wait()

        def scatter_body(ci, _):
            sl = pl.ds(ci * _L, _L)
            plsc.store_scatter(s_v, [p_v[sl]], z_v[sl])
            blkst[sl] = jnp.zeros((_L,), jnp.int32)
            return 0

        lax.fori_loop(0, _NCHUNK, scatter_body, 0)
        blkst[pl.ds(_ROWS, _L)] = jnp.zeros((_L,), jnp.int32)

        def wsum_cf(a_i32, b_i32):
            a = a_i32.astype(jnp.float32)
            b = b_i32.astype(jnp.float32)
            return (b - a + 1.0) * (2.0 * _ROWS - a - b) * 0.5

        def pav_step(i, state):
            sp, tf, ts, tst, cf0 = state
            ii = jnp.full((_L,), i, jnp.int32)
            cfn = plsc.load_gather(s_v, [ii + 1])
            cs0 = jnp.full((_L,), 1.0, jnp.float32)
            cst0 = ii

            def eval_viol(sp_, tf_, ts_, tst_, cf_, cs_, cst_):
                e = jnp.exp(cf_ - tf_)
                tw = wsum_cf(tst_, cst_ - 1)
                cw = wsum_cf(cst_, ii)
                d = cs_ * e * tw - ts_ * cw
                return (sp_ > 0) & (d[0] > 0.0), e

            def cond(st):
                return st[0]

            def merge(st):
                _, e, sp_, tf_, ts_, tst_, cf_, cs_, cst_ = st
                mf, ms, mst = tf_, ts_ + e * cs_, tst_
                nidx = jnp.full((_L,), jnp.maximum(sp_ - 2, 0), jnp.int32)
                ntf = plsc.load_gather(blkf, [nidx])
                nts = plsc.load_gather(blkcs, [nidx])
                ntst = plsc.load_gather(blkst, [nidx])
                sp2 = sp_ - 1
                v2, e2 = eval_viol(sp2, ntf, nts, ntst, mf, ms, mst)
                return (v2, e2, sp2, ntf, nts, ntst, mf, ms, mst)

            v0, e0 = eval_viol(sp, tf, ts, tst, cf0, cs0, cst0)
            _, _, sp, tf, ts, tst, cf, cs, cst = lax.while_loop(
                cond, merge, (v0, e0, sp, tf, ts, tst, cf0, cs0, cst0))
            mpush = mask0 & jnp.full((_L,), sp > 0, jnp.bool_)
            spv = jnp.full((_L,), jnp.maximum(sp - 1, 0), jnp.int32)
            plsc.store_scatter(blkf, [spv], tf, mask=mpush)
            plsc.store_scatter(blkcs, [spv], ts, mask=mpush)
            plsc.store_scatter(blkst, [spv], tst, mask=mpush)
            return (sp + 1, cf, cs, cst, cfn)

        zero_f = jnp.zeros((_L,), jnp.float32)
        zero_i = jnp.zeros((_L,), jnp.int32)
        cf_first = plsc.load_gather(s_v, [jnp.zeros((_L,), jnp.int32)])
        sp, tf, ts, tst, _ = lax.fori_loop(
            0, _ROWS, pav_step, (0, zero_f, zero_f, zero_i, cf_first))

        spv = jnp.full((_L,), sp - 1, jnp.int32)
        plsc.store_scatter(blkf, [spv], tf, mask=mask0)
        plsc.store_scatter(blkcs, [spv], ts, mask=mask0)
        plsc.store_scatter(blkst, [spv], tst, mask=mask0)
        spv = jnp.full((_L,), sp, jnp.int32)
        plsc.store_scatter(blkst, [spv], jnp.full((_L,), _ROWS, jnp.int32),
                           mask=mask0)

        def zero_body(ci, _):
            bid_v[pl.ds(ci * _L, _L)] = jnp.zeros((_L,), jnp.int32)
            return 0

        lax.fori_loop(0, _NCHUNK, zero_body, 0)

        ones = jnp.full((_L,), 1, jnp.int32)

        def mark_body(ci, _):
            idxs = ci * _L + lane
            st = blkst[pl.ds(ci * _L, _L)]
            plsc.store_scatter(bid_v, [st], ones, mask=idxs < sp)
            return 0

        lax.fori_loop(0, _NCHUNK, mark_body, 0)

        def cumsum_body(ci, carry):
            v = bid_v[pl.ds(ci * _L, _L)]
            cs_ = plsc.cumsum(v)
            bid_v[pl.ds(ci * _L, _L)] = cs_ + carry - 1
            return carry + jnp.full((_L,), jnp.max(cs_), jnp.int32)

        lax.fori_loop(0, _NCHUNK, cumsum_body, jnp.zeros((_L,), jnp.int32))

        def out_body(ci, _):
            sl = pl.ds(ci * _L, _L)
            pp = p_v[sl]
            b = plsc.load_gather(bid_v, [pp])
            st = plsc.load_gather(blkst, [b])
            en1 = plsc.load_gather(blkst, [b + 1])
            oa_v[sl] = plsc.load_gather(blkf, [b])
            ocs_v[sl] = plsc.load_gather(blkcs, [b])
            ows_v[sl] = (plsc.load_gather(wc_v, [en1 - 1])
                         - plsc.load_gather(wp_v, [st]))
            return 0

        lax.fori_loop(0, _NCHUNK, out_body, 0)

        o1 = pltpu.async_copy(oa_v, va_hbm.at[row], sem)
        o2 = pltpu.async_copy(ocs_v, vcs_hbm.at[row], sem)
        o3 = pltpu.async_copy(ows_v, vws_hbm.at[row], sem)
        o1.wait(); o2.wait(); o3.wait()


def _pav(z, p, wc, wprev):
    mesh = plsc.VectorSubcoreMesh(core_axis_name="c", subcore_axis_name="s")
    f = pl.kernel(
        _pav_body,
        compiler_params=pltpu.CompilerParams(needs_layout_passes=False),
        out_type=[
            jax.ShapeDtypeStruct((_BATCH, _ROWS), jnp.float32),
            jax.ShapeDtypeStruct((_BATCH, _ROWS), jnp.float32),
            jax.ShapeDtypeStruct((_BATCH, _ROWS), jnp.float32),
        ],
        mesh=mesh,
        scratch_types=[
            pltpu.VMEM((_ROWS,), jnp.float32),
            pltpu.VMEM((_ROWS,), jnp.int32),
            pltpu.VMEM((_ROWS + _L,), jnp.float32),
            pltpu.VMEM((_ROWS,), jnp.float32),
            pltpu.VMEM((_ROWS,), jnp.float32),
            pltpu.VMEM((_ROWS + _L,), jnp.int32),
            pltpu.VMEM((_ROWS,), jnp.int32),
            pltpu.VMEM((_ROWS,), jnp.float32),
            pltpu.VMEM((_ROWS,), jnp.float32),
            pltpu.VMEM((_ROWS,), jnp.float32),
            pltpu.VMEM((_ROWS,), jnp.float32),
            pltpu.VMEM((_ROWS,), jnp.float32),
            pltpu.SemaphoreType.DMA,
        ],
    )
    return f(z, p, wc, wprev)


def _final_body(z_ref, a_ref, cs_ref, ws_ref, o_ref):
    z = z_ref[...]
    lse_s = a_ref[...] + jnp.log(jnp.maximum(cs_ref[...], 1e-38))
    v = lse_s - jnp.log(jnp.maximum(ws_ref[...], 1e-9))
    r = jnp.exp(z - v)
    other = r % 128.0
    o_ref[...] = (r - other) / _CAPACITY + 1.0


def _final(z, va, vcs, vws):
    return pl.pallas_call(
        _final_body,
        out_shape=jax.ShapeDtypeStruct((_BATCH, _ROWS), jnp.float32),
    )(z, va, vcs, vws)


def kernel(table, W1, b1, W2, b2):
    t = table.reshape(_BATCH, _ROWS, _FEAT)
    w_lin = jnp.arange(_ROWS, 0, -1).astype(jnp.float32)
    wc = jnp.cumsum(w_lin)
    wprev = jnp.concatenate([jnp.zeros((1,), jnp.float32), wc[:-1]])
    zcol = _score(t, W1.T, b1.reshape(1, _HID), W2.T, b2.reshape(1, 1))
    z = zcol.reshape(_BATCH, _ROWS)
    zrow = z.reshape(_BATCH, 1, _ROWS)
    p = _positions(zrow, zcol).reshape(_BATCH, _ROWS)
    va, vcs, vws = _pav(z, p, wc, wprev)
    out = _final(z, va, vcs, vws)
    return out.reshape(_BATCH, _ROWS, 1)


# speedup vs baseline: 1.5802x; 1.0550x over previous
import functools

import jax
import jax.numpy as jnp
from jax import lax
from jax.experimental import pallas as pl
from jax.experimental.pallas import tpu as pltpu
from jax.experimental.pallas import tpu_sc as plsc

_COL_NUM = 12
_DMODEL = 16
_CAPACITY = 128
_BATCH = 8
_ROWS = 1024
_FEAT = _COL_NUM * _DMODEL
_HID = 32
_L = 16
_NCHUNK = _ROWS // _L


def _score_body(t_ref, w1t_ref, b1_ref, w2_ref, b2_ref, z_ref):
    t = t_ref[0]
    h = jnp.dot(t, w1t_ref[...], preferred_element_type=jnp.float32) + b1_ref[...]
    h = jnp.maximum(h, 0.0)
    s2 = jnp.dot(h, w2_ref[...], preferred_element_type=jnp.float32) + b2_ref[0, 0]
    mn = jnp.min(s2)
    mx = jnp.max(s2)
    z_ref[0] = (s2 - mn) / (mx - mn + 0.1) * 100.0


def _score(t, w1t, b1, w2, b2):
    return pl.pallas_call(
        _score_body,
        grid=(_BATCH,),
        in_specs=[
            pl.BlockSpec((1, _ROWS, _FEAT), lambda i: (i, 0, 0)),
            pl.BlockSpec((_FEAT, _HID), lambda i: (0, 0)),
            pl.BlockSpec((1, _HID), lambda i: (0, 0)),
            pl.BlockSpec((_HID, 1), lambda i: (0, 0)),
            pl.BlockSpec((1, 1), lambda i: (0, 0)),
        ],
        out_specs=pl.BlockSpec((1, _ROWS, 1), lambda i: (i, 0, 0)),
        out_shape=jax.ShapeDtypeStruct((_BATCH, _ROWS, 1), jnp.float32),
    )(t, w1t, b1, w2, b2)


_CCH = 1024
_NJ = _ROWS // _CCH


def _pos_body(zrow_ref, zcol_ref, p_ref):
    j = pl.program_id(1)
    zr = zrow_ref[0]
    zc = zcol_ref[0]
    gt = zr > zc
    ic = lax.broadcasted_iota(jnp.int32, (_CCH, _ROWS), 0) + j * _CCH
    jc = lax.broadcasted_iota(jnp.int32, (_CCH, _ROWS), 1)
    tie = (zr == zc) & (jc < ic)
    m = jnp.where(gt | tie, 1.0, 0.0)
    cnt = jnp.dot(m, jnp.ones((_ROWS, 1), jnp.float32),
                  preferred_element_type=jnp.float32)
    p_ref[0] = cnt.astype(jnp.int32)


def _positions(zrow, zcol):
    return pl.pallas_call(
        _pos_body,
        grid=(_BATCH, _NJ),
        in_specs=[
            pl.BlockSpec((1, 1, _ROWS), lambda i, j: (i, 0, 0)),
            pl.BlockSpec((1, _CCH, 1), lambda i, j: (i, j, 0)),
        ],
        out_specs=pl.BlockSpec((1, _CCH, 1), lambda i, j: (i, j, 0)),
        out_shape=jax.ShapeDtypeStruct((_BATCH, _ROWS, 1), jnp.int32),
    )(zrow, zcol)


def _pav_body(z_hbm, p_hbm, wc_hbm, wp_hbm, va_hbm, vcs_hbm, vws_hbm,
              z_v, p_v, s_v, blkf, blkcs, blkst, bid_v, wc_v, wp_v,
              oa_v, ocs_v, ows_v, sem):
    cid = lax.axis_index("c")
    sid = lax.axis_index("s")
    lane = lax.iota(jnp.int32, _L)
    mask0 = lane == 0

    @pl.when(sid < _BATCH // 2)
    def _():
        row = cid * (_BATCH // 2) + sid
        c1 = pltpu.async_copy(z_hbm.at[row], z_v, sem)
        c2 = pltpu.async_copy(p_hbm.at[row], p_v, sem)
        c3 = pltpu.async_copy(wc_hbm, wc_v, sem)
        c4 = pltpu.async_copy(wp_hbm, wp_v, sem)
        c1.wait(); c2.wait(); c3.wait(); c4.wait()

        def scatter_body(ci, _):
            sl = pl.ds(ci * _L, _L)
            plsc.store_scatter(s_v, [p_v[sl]], z_v[sl])
            blkst[sl] = jnp.zeros((_L,), jnp.int32)
            return 0

        lax.fori_loop(0, _NCHUNK, scatter_body, 0)
        blkst[pl.ds(_ROWS, _L)] = jnp.zeros((_L,), jnp.int32)

        def wsum_cf(a_i32, b_i32):
            a = a_i32.astype(jnp.float32)
            b = b_i32.astype(jnp.float32)
            return (b - a + 1.0) * (2.0 * _ROWS - a - b) * 0.5

        def pav_step(i, state):
            sp, tf, ts, tst, cf0 = state
            ii = jnp.full((_L,), i, jnp.int32)
            cfn = plsc.load_gather(s_v, [ii + 1])
            cs0 = jnp.full((_L,), 1.0, jnp.float32)
            cst0 = ii

            def eval_viol(sp_, tf_, ts_, tst_, cf_, cs_, cst_):
                e = jnp.exp(cf_ - tf_)
                tw = wsum_cf(tst_, cst_ - 1)
                cw = wsum_cf(cst_, ii)
                d = cs_ * e * tw - ts_ * cw
                return (sp_ > 0) & (d[0] > 0.0), e

            def cond(st):
                return st[0]

            def merge(st):
                _, e, sp_, tf_, ts_, tst_, cf_, cs_, cst_ = st
                mf, ms, mst = tf_, ts_ + e * cs_, tst_
                nidx = jnp.full((_L,), jnp.maximum(sp_ - 2, 0), jnp.int32)
                ntf = plsc.load_gather(blkf, [nidx])
                nts = plsc.load_gather(blkcs, [nidx])
                ntst = plsc.load_gather(blkst, [nidx])
                sp2 = sp_ - 1
                v2, e2 = eval_viol(sp2, ntf, nts, ntst, mf, ms, mst)
                return (v2, e2, sp2, ntf, nts, ntst, mf, ms, mst)

            v0, e0 = eval_viol(sp, tf, ts, tst, cf0, cs0, cst0)
            _, _, sp, tf, ts, tst, cf, cs, cst = lax.while_loop(
                cond, merge, (v0, e0, sp, tf, ts, tst, cf0, cs0, cst0))
            mpush = mask0 & jnp.full((_L,), sp > 0, jnp.bool_)
            spv = jnp.full((_L,), jnp.maximum(sp - 1, 0), jnp.int32)
            plsc.store_scatter(blkf, [spv], tf, mask=mpush)
            plsc.store_scatter(blkcs, [spv], ts, mask=mpush)
            plsc.store_scatter(blkst, [spv], tst, mask=mpush)
            return (sp + 1, cf, cs, cst, cfn)

        zero_f = jnp.zeros((_L,), jnp.float32)
        zero_i = jnp.zeros((_L,), jnp.int32)
        cf_first = plsc.load_gather(s_v, [jnp.zeros((_L,), jnp.int32)])
        sp, tf, ts, tst, _ = lax.fori_loop(
            0, _ROWS, pav_step, (0, zero_f, zero_f, zero_i, cf_first))

        spv = jnp.full((_L,), sp - 1, jnp.int32)
        plsc.store_scatter(blkf, [spv], tf, mask=mask0)
        plsc.store_scatter(blkcs, [spv], ts, mask=mask0)
        plsc.store_scatter(blkst, [spv], tst, mask=mask0)
        spv = jnp.full((_L,), sp, jnp.int32)
        plsc.store_scatter(blkst, [spv], jnp.full((_L,), _ROWS, jnp.int32),
                           mask=mask0)

        def zero_body(ci, _):
            bid_v[pl.ds(ci * _L, _L)] = jnp.zeros((_L,), jnp.int32)
            return 0

        lax.fori_loop(0, _NCHUNK, zero_body, 0)

        ones = jnp.full((_L,), 1, jnp.int32)

        def mark_body(ci, _):
            idxs = ci * _L + lane
            st = blkst[pl.ds(ci * _L, _L)]
            plsc.store_scatter(bid_v, [st], ones, mask=idxs < sp)
            return 0

        lax.fori_loop(0, _NCHUNK, mark_body, 0)

        def cumsum_body(ci, carry):
            v = bid_v[pl.ds(ci * _L, _L)]
            cs_ = plsc.cumsum(v)
            bid_v[pl.ds(ci * _L, _L)] = cs_ + carry - 1
            return carry + jnp.full((_L,), jnp.max(cs_), jnp.int32)

        lax.fori_loop(0, _NCHUNK, cumsum_body, jnp.zeros((_L,), jnp.int32))

        def out_body(ci, _):
            sl = pl.ds(ci * _L, _L)
            pp = p_v[sl]
            b = plsc.load_gather(bid_v, [pp])
            st = plsc.load_gather(blkst, [b])
            en1 = plsc.load_gather(blkst, [b + 1])
            oa_v[sl] = plsc.load_gather(blkf, [b])
            ocs_v[sl] = plsc.load_gather(blkcs, [b])
            ows_v[sl] = (plsc.load_gather(wc_v, [en1 - 1])
                         - plsc.load_gather(wp_v, [st]))
            return 0

        lax.fori_loop(0, _NCHUNK, out_body, 0)

        o1 = pltpu.async_copy(oa_v, va_hbm.at[row], sem)
        o2 = pltpu.async_copy(ocs_v, vcs_hbm.at[row], sem)
        o3 = pltpu.async_copy(ows_v, vws_hbm.at[row], sem)
        o1.wait(); o2.wait(); o3.wait()


def _pav(z, p, wc, wprev):
    mesh = plsc.VectorSubcoreMesh(core_axis_name="c", subcore_axis_name="s")
    f = pl.kernel(
        _pav_body,
        compiler_params=pltpu.CompilerParams(needs_layout_passes=False),
        out_type=[
            jax.ShapeDtypeStruct((_BATCH, _ROWS), jnp.float32),
            jax.ShapeDtypeStruct((_BATCH, _ROWS), jnp.float32),
            jax.ShapeDtypeStruct((_BATCH, _ROWS), jnp.float32),
        ],
        mesh=mesh,
        scratch_types=[
            pltpu.VMEM((_ROWS,), jnp.float32),
            pltpu.VMEM((_ROWS,), jnp.int32),
            pltpu.VMEM((_ROWS + _L,), jnp.float32),
            pltpu.VMEM((_ROWS,), jnp.float32),
            pltpu.VMEM((_ROWS,), jnp.float32),
            pltpu.VMEM((_ROWS + _L,), jnp.int32),
            pltpu.VMEM((_ROWS,), jnp.int32),
            pltpu.VMEM((_ROWS,), jnp.float32),
            pltpu.VMEM((_ROWS,), jnp.float32),
            pltpu.VMEM((_ROWS,), jnp.float32),
            pltpu.VMEM((_ROWS,), jnp.float32),
            pltpu.VMEM((_ROWS,), jnp.float32),
            pltpu.SemaphoreType.DMA,
        ],
    )
    return f(z, p, wc, wprev)


def _final_body(z_ref, a_ref, cs_ref, ws_ref, o_ref):
    z = z_ref[...]
    lse_s = a_ref[...] + jnp.log(jnp.maximum(cs_ref[...], 1e-38))
    v = lse_s - jnp.log(jnp.maximum(ws_ref[...], 1e-9))
    r = jnp.exp(z - v)
    other = r % 128.0
    o_ref[...] = (r - other) / _CAPACITY + 1.0


def _final(z, va, vcs, vws):
    return pl.pallas_call(
        _final_body,
        out_shape=jax.ShapeDtypeStruct((_BATCH, _ROWS), jnp.float32),
    )(z, va, vcs, vws)


def kernel(table, W1, b1, W2, b2):
    t = table.reshape(_BATCH, _ROWS, _FEAT)
    w_lin = jnp.arange(_ROWS, 0, -1).astype(jnp.float32)
    wc = jnp.cumsum(w_lin)
    wprev = jnp.concatenate([jnp.zeros((1,), jnp.float32), wc[:-1]])
    zcol = _score(t, W1.T, b1.reshape(1, _HID), W2.T, b2.reshape(1, 1))
    z = zcol.reshape(_BATCH, _ROWS)
    zrow = z.reshape(_BATCH, 1, _ROWS)
    p = _positions(zrow, zcol).reshape(_BATCH, _ROWS)
    va, vcs, vws = _pav(z, p, wc, wprev)
    out = _final(z, va, vcs, vws)
    return out.reshape(_BATCH, _ROWS, 1)
